# Initial kernel scaffold; baseline (speedup 1.0000x reference)
import jax
import jax.numpy as jnp
from jax.experimental import pallas as pl


def kernel(x, edge_index, Wl1, bl1, Wr1, Wl2, bl2, Wr2):
    raise NotImplementedError("write your pallas kernel here")


# speedup vs baseline: 3.0095x; 3.0095x over previous
import functools

import jax
import jax.numpy as jnp
from jax import lax
from jax.experimental import pallas as pl
from jax.experimental.pallas import tpu as pltpu
from jax.experimental.pallas import tpu_sc as plsc

N = 10000
D = 128
E = 320000
NC = 2
NS = 16
CHUNK = 128
NCHUNK = 80
E_PAD = NC * NS * NCHUNK * CHUNK
NPAD = 10240
ROWS_PT = NPAD // NS
DUMMY_DST = NPAD - 1


def _seg_body(x_hbm, src_hbm, dst_hbm, psum_hbm, pcnt_hbm,
              acc_sh, cnt_sh, src_v, dst_v, rows_v, ones_v, czv, sem):
    c = lax.axis_index("c")
    s = lax.axis_index("s")

    pltpu.sync_copy(src_hbm.at[c, s], src_v)
    pltpu.sync_copy(dst_hbm.at[c, s], dst_v)

    zero16 = jnp.zeros((16,), jnp.float32)
    one16 = jnp.ones((16,), jnp.float32)

    def _zrows(i, _):
        rows_v[i // 8, pl.ds((i % 8) * 16, 16)] = zero16
        return 0
    lax.fori_loop(0, CHUNK * 8, _zrows, 0)

    def _ones(i, _):
        ones_v[pl.ds(i * 16, 16)] = one16
        return 0
    lax.fori_loop(0, CHUNK // 16, _ones, 0)

    def _zc(i, _):
        czv[pl.ds(i * 16, 16)] = zero16
        return 0
    lax.fori_loop(0, ROWS_PT // 16, _zc, 0)

    def _zslab(k, _):
        pltpu.sync_copy(rows_v, acc_sh.at[pl.ds(s * ROWS_PT + k * CHUNK, CHUNK)])
        return 0
    lax.fori_loop(0, ROWS_PT // CHUNK, _zslab, 0)
    pltpu.sync_copy(czv, cnt_sh.at[pl.ds(s * ROWS_PT, ROWS_PT)])
    plsc.subcore_barrier()

    def _chunk(j, _):
        pltpu.async_copy(x_hbm.at[src_v.at[j]], rows_v, sem).wait()
        pltpu.sync_copy(rows_v, acc_sh.at[dst_v.at[j]], add=True)
        pltpu.sync_copy(ones_v, cnt_sh.at[dst_v.at[j]], add=True)
        return 0
    lax.fori_loop(0, NCHUNK, _chunk, 0)
    plsc.subcore_barrier()

    pltpu.sync_copy(acc_sh.at[pl.ds(s * ROWS_PT, ROWS_PT)],
                    psum_hbm.at[c, pl.ds(s * ROWS_PT, ROWS_PT)])
    pltpu.sync_copy(cnt_sh.at[pl.ds(s * ROWS_PT, ROWS_PT)],
                    pcnt_hbm.at[c, pl.ds(s * ROWS_PT, ROWS_PT)])


_seg_sum = pl.kernel(
    _seg_body,
    out_type=(jax.ShapeDtypeStruct((NC, NPAD, D), jnp.float32),
              jax.ShapeDtypeStruct((NC, NPAD), jnp.float32)),
    mesh=plsc.VectorSubcoreMesh(core_axis_name="c", subcore_axis_name="s",
                                num_cores=NC, num_subcores=NS),
    scratch_types=[
        pltpu.VMEM_SHARED((NPAD, D), jnp.float32),
        pltpu.VMEM_SHARED((NPAD,), jnp.float32),
        pltpu.VMEM((NCHUNK, CHUNK), jnp.int32),
        pltpu.VMEM((NCHUNK, CHUNK), jnp.int32),
        pltpu.VMEM((CHUNK, D), jnp.float32),
        pltpu.VMEM((CHUNK,), jnp.float32),
        pltpu.VMEM((ROWS_PT,), jnp.float32),
        pltpu.SemaphoreType.DMA,
    ],
)


BLK = 1280


def _layer_body(relu, p_ref, c_ref, x_ref, wl_ref, bl_ref, wr_ref, o_ref):
    i = pl.program_id(0)
    cnt = c_ref[0, pl.ds(i * BLK, BLK)] + c_ref[1, pl.ds(i * BLK, BLK)]
    recip = 1.0 / jnp.maximum(cnt, 1.0)
    mean = (p_ref[0] + p_ref[1]) * recip[:, None]
    t = (jnp.dot(mean, wl_ref[...], preferred_element_type=jnp.float32)
         + jnp.dot(x_ref[...], wr_ref[...], preferred_element_type=jnp.float32)
         + bl_ref[...])
    o_ref[...] = jnp.maximum(t, 0.0) if relu else t


def _make_layer(relu):
    return pl.pallas_call(
        functools.partial(_layer_body, relu),
        grid=(NPAD // BLK,),
        in_specs=[
            pl.BlockSpec((NC, BLK, D), lambda i: (0, i, 0)),
            pl.BlockSpec((NC, NPAD), lambda i: (0, 0)),
            pl.BlockSpec((BLK, D), lambda i: (i, 0)),
            pl.BlockSpec((D, D), lambda i: (0, 0)),
            pl.BlockSpec((1, D), lambda i: (0, 0)),
            pl.BlockSpec((D, D), lambda i: (0, 0)),
        ],
        out_specs=pl.BlockSpec((BLK, D), lambda i: (i, 0)),
        out_shape=jax.ShapeDtypeStruct((NPAD, D), jnp.float32),
    )


_layer_relu = _make_layer(True)
_layer_lin = _make_layer(False)


def kernel(x, edge_index, Wl1, bl1, Wr1, Wl2, bl2, Wr2):
    pad = E_PAD - E
    src = jnp.concatenate(
        [edge_index[0], jnp.zeros((pad,), jnp.int32)]).reshape(
            NC, NS, NCHUNK, CHUNK)
    dst = jnp.concatenate(
        [edge_index[1], jnp.full((pad,), DUMMY_DST, jnp.int32)]).reshape(
            NC, NS, NCHUNK, CHUNK)
    p1, c1 = _seg_sum(x, src, dst)
    h = _layer_relu(p1, c1, x, Wl1, bl1.reshape(1, D), Wr1)
    p2, c2 = _seg_sum(h, src, dst)
    out = _layer_lin(p2, c2, h, Wl2, bl2.reshape(1, D), Wr2)
    return out[:N]


# speedup vs baseline: 3.5298x; 1.1729x over previous
import functools

import jax
import jax.numpy as jnp
from jax import lax
from jax.experimental import pallas as pl
from jax.experimental.pallas import tpu as pltpu
from jax.experimental.pallas import tpu_sc as plsc

N = 10000
D = 128
E = 320000
NC = 2
NS = 16
CHUNK = 128
NCHUNK = 80
NHALF = 2
HCH = NCHUNK // NHALF
HPAIR = HCH // 2
E_PAD = NC * NS * NCHUNK * CHUNK
NPAD = 10240
ROWS_PT = NPAD // NS
DUMMY_DST = NPAD - 1


def _seg_body(with_counts, x_hbm, src_hbm, dst_hbm, psum_hbm, *rest):
    if with_counts:
        (pcnt_hbm, acc_sh, cnt_sh, src_v, dst_v, rows0, rows1, ones_v, czv,
         g0, g1, s0, s1) = rest
    else:
        (acc_sh, src_v, dst_v, rows0, rows1, g0, g1, s0, s1) = rest
    c = lax.axis_index("c")
    s = lax.axis_index("s")

    zero16 = jnp.zeros((16,), jnp.float32)

    def _zrows(i, _):
        rows0[i // 8, pl.ds((i % 8) * 16, 16)] = zero16
        return 0
    lax.fori_loop(0, CHUNK * 8, _zrows, 0)

    if with_counts:
        one16 = jnp.ones((16,), jnp.float32)

        def _ones(i, _):
            ones_v[pl.ds(i * 16, 16)] = one16
            return 0
        lax.fori_loop(0, CHUNK // 16, _ones, 0)

        def _zc(i, _):
            czv[pl.ds(i * 16, 16)] = zero16
            return 0
        lax.fori_loop(0, ROWS_PT // 16, _zc, 0)

    def _zslab(k, _):
        pltpu.sync_copy(rows0, acc_sh.at[pl.ds(s * ROWS_PT + k * CHUNK, CHUNK)])
        return 0
    lax.fori_loop(0, ROWS_PT // CHUNK, _zslab, 0)
    if with_counts:
        pltpu.sync_copy(czv, cnt_sh.at[pl.ds(s * ROWS_PT, ROWS_PT)])
    plsc.subcore_barrier()

    def _start_gather(j, buf, sem):
        pltpu.async_copy(x_hbm.at[src_v.at[j]], buf, sem)

    def _wait_gather(j, buf, sem):
        pltpu.make_async_copy(x_hbm.at[src_v.at[j]], buf, sem).wait()

    def _start_scat(j, buf, sem):
        pltpu.async_copy(buf, acc_sh.at[dst_v.at[j]], sem, add=True)

    def _wait_scat(j, buf, sem):
        pltpu.make_async_copy(buf, acc_sh.at[dst_v.at[j]], sem).wait()

    def _counts(j):
        if with_counts:
            pltpu.sync_copy(ones_v, cnt_sh.at[dst_v.at[j]], add=True)

    def _half(h, _):
        pltpu.sync_copy(src_hbm.at[c, s, pl.ds(h * HCH, HCH)], src_v)
        pltpu.sync_copy(dst_hbm.at[c, s, pl.ds(h * HCH, HCH)], dst_v)
        _start_gather(0, rows0, g0)

        def _pair(j, _):
            a = 2 * j
            b = a + 1
            _wait_gather(a, rows0, g0)

            @pl.when(j > 0)
            def _():
                _wait_scat(b - 2, rows1, s1)
            _start_gather(b, rows1, g1)
            _start_scat(a, rows0, s0)
            _counts(a)
            _wait_gather(b, rows1, g1)
            _wait_scat(a, rows0, s0)

            @pl.when(j < HPAIR - 1)
            def _():
                _start_gather(a + 2, rows0, g0)
            _start_scat(b, rows1, s1)
            _counts(b)
            return 0
        lax.fori_loop(0, HPAIR, _pair, 0)
        _wait_scat(HCH - 1, rows1, s1)
        return 0
    lax.fori_loop(0, NHALF, _half, 0)
    plsc.subcore_barrier()

    pltpu.sync_copy(acc_sh.at[pl.ds(s * ROWS_PT, ROWS_PT)],
                    psum_hbm.at[c, pl.ds(s * ROWS_PT, ROWS_PT)])
    if with_counts:
        pltpu.sync_copy(cnt_sh.at[pl.ds(s * ROWS_PT, ROWS_PT)],
                        pcnt_hbm.at[c, pl.ds(s * ROWS_PT, ROWS_PT)])


def _make_seg_sum(with_counts):
    out_type = [jax.ShapeDtypeStruct((NC, NPAD, D), jnp.float32)]
    scratch = [pltpu.VMEM_SHARED((NPAD, D), jnp.float32)]
    if with_counts:
        out_type.append(jax.ShapeDtypeStruct((NC, NPAD), jnp.float32))
        scratch.append(pltpu.VMEM_SHARED((NPAD,), jnp.float32))
    scratch += [
        pltpu.VMEM((HCH, CHUNK), jnp.int32),
        pltpu.VMEM((HCH, CHUNK), jnp.int32),
        pltpu.VMEM((CHUNK, D), jnp.float32),
        pltpu.VMEM((CHUNK, D), jnp.float32),
    ]
    if with_counts:
        scratch += [
            pltpu.VMEM((CHUNK,), jnp.float32),
            pltpu.VMEM((ROWS_PT,), jnp.float32),
        ]
    scratch += [pltpu.SemaphoreType.DMA] * 4
    return pl.kernel(
        functools.partial(_seg_body, with_counts),
        out_type=tuple(out_type),
        mesh=plsc.VectorSubcoreMesh(core_axis_name="c", subcore_axis_name="s",
                                    num_cores=NC, num_subcores=NS),
        scratch_types=scratch,
    )


_seg_sum_cnt = _make_seg_sum(True)
_seg_sum = _make_seg_sum(False)


BLK = 1280


def _layer_body(relu, p_ref, c_ref, x_ref, wl_ref, bl_ref, wr_ref, o_ref):
    i = pl.program_id(0)
    cnt = c_ref[0, pl.ds(i * BLK, BLK)] + c_ref[1, pl.ds(i * BLK, BLK)]
    recip = 1.0 / jnp.maximum(cnt, 1.0)
    mean = (p_ref[0] + p_ref[1]) * recip[:, None]
    t = (jnp.dot(mean, wl_ref[...], preferred_element_type=jnp.float32)
         + jnp.dot(x_ref[...], wr_ref[...], preferred_element_type=jnp.float32)
         + bl_ref[...])
    o_ref[...] = jnp.maximum(t, 0.0) if relu else t


def _make_layer(relu):
    return pl.pallas_call(
        functools.partial(_layer_body, relu),
        grid=(NPAD // BLK,),
        in_specs=[
            pl.BlockSpec((NC, BLK, D), lambda i: (0, i, 0)),
            pl.BlockSpec((NC, NPAD), lambda i: (0, 0)),
            pl.BlockSpec((BLK, D), lambda i: (i, 0)),
            pl.BlockSpec((D, D), lambda i: (0, 0)),
            pl.BlockSpec((1, D), lambda i: (0, 0)),
            pl.BlockSpec((D, D), lambda i: (0, 0)),
        ],
        out_specs=pl.BlockSpec((BLK, D), lambda i: (i, 0)),
        out_shape=jax.ShapeDtypeStruct((NPAD, D), jnp.float32),
    )


_layer_relu = _make_layer(True)
_layer_lin = _make_layer(False)


def kernel(x, edge_index, Wl1, bl1, Wr1, Wl2, bl2, Wr2):
    pad = E_PAD - E
    src = jnp.concatenate(
        [edge_index[0], jnp.zeros((pad,), jnp.int32)]).reshape(
            NC, NS, NCHUNK, CHUNK)
    dst = jnp.concatenate(
        [edge_index[1], jnp.full((pad,), DUMMY_DST, jnp.int32)]).reshape(
            NC, NS, NCHUNK, CHUNK)
    p1, c1 = _seg_sum_cnt(x, src, dst)
    h = _layer_relu(p1, c1, x, Wl1, bl1.reshape(1, D), Wr1)
    (p2,) = _seg_sum(h, src, dst)
    out = _layer_lin(p2, c1, h, Wl2, bl2.reshape(1, D), Wr2)
    return out[:N]
